# baseline (speedup 1.0000x reference)
import jax
import jax.numpy as jnp
from jax import lax
from jax.experimental import pallas as pl
from jax.experimental.pallas import tpu as pltpu
from jax.experimental.pallas import tpu_sc as plsc

B = 1024
D = 128
C = 256
CAP = 64
TAUC = 1.0
CK = 32
NCHUNK = C // CK
W = CK * CAP

_f32 = jnp.float32
_bf16 = jnp.bfloat16


PAD = 16


def _route_kernel(cls_hbm, counts_out, ranks_out, cls_v, cnt_v, rk_v):
    cid = lax.axis_index("c")
    sid = lax.axis_index("s")

    @pl.when(jnp.logical_and(cid == 0, sid == 0))
    def _():
        pltpu.sync_copy(cls_hbm, cls_v.at[pl.ds(0, B)])
        lane0 = lax.iota(jnp.int32, 16) == 0
        for v in range((C + PAD) // 16):
            cnt_v[pl.ds(v * 16, 16)] = jnp.zeros((16,), jnp.int32)

        def body(j, carry):
            cj = cls_v[pl.ds(j, 16)][0]
            cvec = cnt_v[pl.ds(cj, 16)]
            rj = cvec[0]
            rvec = rk_v[pl.ds(j, 16)]
            rk_v[pl.ds(j, 16)] = jnp.where(lane0, rj, rvec)
            cnt_v[pl.ds(cj, 16)] = jnp.where(lane0, rj + 1, cvec)
            return carry

        lax.fori_loop(0, B, body, 0)
        pltpu.sync_copy(cnt_v.at[pl.ds(0, C)], counts_out)
        pltpu.sync_copy(rk_v.at[pl.ds(0, B)], ranks_out)


def _route(classes):
    k = pl.kernel(
        _route_kernel,
        mesh=plsc.VectorSubcoreMesh(core_axis_name="c", subcore_axis_name="s"),
        out_type=[jax.ShapeDtypeStruct((C,), jnp.int32),
                  jax.ShapeDtypeStruct((B,), jnp.int32)],
        scratch_types=[pltpu.VMEM((B + PAD,), jnp.int32),
                       pltpu.VMEM((C + PAD,), jnp.int32),
                       pltpu.VMEM((B + PAD,), jnp.int32)],
    )
    return k(classes)


def _loss_kernel(x_ref, xb_ref, mem_ref, clsc_ref, cnt_ref, rk_ref, out_ref):
    x = x_ref[:, :]
    xb = xb_ref[:, :]
    cls_col = clsc_ref[:, :]
    counts_row = cnt_ref[:, :]
    r_col = rk_ref[:, :]

    nrm = jnp.sqrt(jnp.sum(x * x, axis=1, keepdims=True))
    feat = (x / jnp.maximum(nrm, 1e-12)).astype(_bf16)

    cls_iota = jax.lax.broadcasted_iota(jnp.int32, (B, C), 1)
    onehotb = (cls_col == cls_iota).astype(_bf16)
    onehotf = onehotb.astype(_f32)
    incl_col = (r_col < CAP).astype(_bf16)

    cm4_col = cls_col - (cls_col // 4) * 4
    bidx_g = jnp.where(r_col < CAP, cm4_col * CAP + r_col, C)
    gw = ((bidx_g == jax.lax.broadcasted_iota(jnp.int32, (B, C), 1))
          .astype(_bf16))
    qw = onehotb * incl_col

    G = jax.lax.dot_general(feat, xb, (((1,), (1,)), ((), ())),
                            preferred_element_type=_f32)
    Gb = G.astype(_bf16)
    eG = jnp.exp(Gb)
    TG = jax.lax.dot_general(eG, incl_col, (((1,), (0,)), ((), ())),
                             preferred_element_type=_f32)
    GEX = jax.lax.dot_general(eG, gw, (((1,), (0,)), ((), ())),
                              preferred_element_type=_f32)
    POSG = jax.lax.dot_general(Gb, qw, (((1,), (0,)), ((), ())),
                               preferred_element_type=_f32)

    colw = jax.lax.broadcasted_iota(jnp.int32, (W, 1), 0)
    lcls_w = colw // CAP
    t_w = colw - lcls_w * CAP
    gm4_w = lcls_w - (lcls_w // 4) * 4
    oc = (jax.lax.broadcasted_iota(jnp.int32, (W, CK), 0) // CAP ==
          jax.lax.broadcasted_iota(jnp.int32, (W, CK), 1))
    ocb = oc.astype(_bf16)
    ocf = oc.astype(_f32)
    kiota = jax.lax.broadcasted_iota(jnp.int32, (W, C), 1)

    T = TG
    EX = GEX
    posacc = jnp.sum(POSG * onehotf, axis=1, keepdims=True)

    for k in range(NCHUNK):
        mb = mem_ref[pl.ds(k * W, W), :]
        Mc = jax.lax.dot_general(feat, mb, (((1,), (1,)), ((), ())),
                                 preferred_element_type=_f32).astype(_bf16)
        eM = jnp.exp(Mc)

        countsc = counts_row[:, k * CK:(k + 1) * CK]
        ccol_w = jax.lax.dot_general(ocf, countsc,
                                     (((1,), (1,)), ((), ())),
                                     preferred_element_type=_f32)
        tpc_w = t_w + ccol_w.astype(jnp.int32)
        keep_w = (tpc_w < CAP).astype(_bf16)
        bidx_w = jnp.where(tpc_w < CAP, gm4_w * CAP + tpc_w, C)
        exw = (bidx_w == kiota).astype(_bf16)
        kwc = ocb * keep_w

        T = T + jax.lax.dot_general(eM, keep_w, (((1,), (0,)), ((), ())),
                                    preferred_element_type=_f32)
        EX = EX + jax.lax.dot_general(eM, exw, (((1,), (0,)), ((), ())),
                                      preferred_element_type=_f32)
        posc = jax.lax.dot_general(Mc, kwc, (((1,), (0,)), ((), ())),
                                   preferred_element_type=_f32)
        posacc = posacc + jnp.sum(
            posc * onehotf[:, k * CK:(k + 1) * CK], axis=1, keepdims=True)

    ex_i = jnp.sum(EX * onehotf, axis=1, keepdims=True)
    pos = posacc * (1.0 / CAP)
    Tn = T - ex_i
    denom = jnp.exp(pos) + Tn
    lossv = jnp.log(denom) - pos
    out_ref[:, :] = jnp.reshape(jnp.sum(lossv) * (1.0 / B), (1, 1))


def kernel(x, memory, classes):
    counts_i, ranks_i = _route(classes)
    counts_row = counts_i.astype(jnp.float32).reshape(1, C)
    r2d = ranks_i.reshape(B, 1)
    mem_flat = memory.reshape(C * CAP, D).astype(_bf16)
    xbf = x.astype(_bf16)
    cls2d = classes.reshape(B, 1)
    out = pl.pallas_call(
        _loss_kernel,
        out_shape=jax.ShapeDtypeStruct((1, 1), jnp.float32),
    )(x, xbf, mem_flat, cls2d, counts_row, r2d)
    return out[0, 0]


# speedup vs baseline: 1.2643x; 1.2643x over previous
import jax
import jax.numpy as jnp
from jax import lax
from jax.experimental import pallas as pl
from jax.experimental.pallas import tpu as pltpu
from jax.experimental.pallas import tpu_sc as plsc

B = 1024
D = 128
C = 256
CAP = 64
TAUC = 1.0
CK = 32
NCHUNK = C // CK
W = CK * CAP

_f32 = jnp.float32
_bf16 = jnp.bfloat16


PAD = 16
NW = 16
BW = B // NW


def _route_kernel(cls_hbm, counts_out, ranks_out, stage_hbm,
                  cls_v, cnt_v, rk_v, allt_v, pref_v):
    cid = lax.axis_index("c")
    sid = lax.axis_index("s")

    @pl.when(cid == 0)
    def _():
        base = sid * BW
        pltpu.sync_copy(cls_hbm.at[pl.ds(base, BW)], cls_v.at[pl.ds(0, BW)])
        lane0 = lax.iota(jnp.int32, 16) == 0
        for v in range((C + PAD) // 16):
            cnt_v[pl.ds(v * 16, 16)] = jnp.zeros((16,), jnp.int32)

        def body(j, carry):
            cj = cls_v[pl.ds(j, 16)][0]
            cvec = cnt_v[pl.ds(cj, 16)]
            rj = cvec[0]
            rvec = rk_v[pl.ds(j, 16)]
            rk_v[pl.ds(j, 16)] = jnp.where(lane0, rj, rvec)
            cnt_v[pl.ds(cj, 16)] = jnp.where(lane0, rj + 1, cvec)
            return carry

        lax.fori_loop(0, BW, body, 0)
        pltpu.sync_copy(cnt_v.at[pl.ds(0, C)], stage_hbm.at[pl.ds(sid * C, C)])
        plsc.subcore_barrier()
        pltpu.sync_copy(stage_hbm, allt_v)
        for v in range(C // 16):
            pref_v[pl.ds(v * 16, 16)] = jnp.zeros((16,), jnp.int32)

        def merge(w, carry):
            @pl.when(w < sid)
            def _():
                for v in range(C // 16):
                    pref_v[pl.ds(v * 16, 16)] = (
                        pref_v[pl.ds(v * 16, 16)]
                        + allt_v[pl.ds(w * C + v * 16, 16)])
            return carry

        lax.fori_loop(0, NW, merge, 0)
        def addoff(j, carry):
            cj = cls_v[pl.ds(j, 16)][0]
            off = pref_v[pl.ds(cj, 16)][0]
            rvec = rk_v[pl.ds(j, 16)]
            rk_v[pl.ds(j, 16)] = jnp.where(lane0, rvec[0] + off, rvec)
            return carry

        lax.fori_loop(0, BW, addoff, 0)
        pltpu.sync_copy(rk_v.at[pl.ds(0, BW)], ranks_out.at[pl.ds(base, BW)])

        @pl.when(sid == NW - 1)
        def _():
            for v in range(C // 16):
                pref_v[pl.ds(v * 16, 16)] = (
                    pref_v[pl.ds(v * 16, 16)] + cnt_v[pl.ds(v * 16, 16)])
            pltpu.sync_copy(pref_v.at[pl.ds(0, C)], counts_out)


def _route(classes):
    k = pl.kernel(
        _route_kernel,
        mesh=plsc.VectorSubcoreMesh(core_axis_name="c", subcore_axis_name="s"),
        out_type=[jax.ShapeDtypeStruct((C,), jnp.int32),
                  jax.ShapeDtypeStruct((B,), jnp.int32),
                  jax.ShapeDtypeStruct((NW * C,), jnp.int32)],
        scratch_types=[pltpu.VMEM((BW + PAD,), jnp.int32),
                       pltpu.VMEM((C + PAD,), jnp.int32),
                       pltpu.VMEM((BW + PAD,), jnp.int32),
                       pltpu.VMEM((NW * C,), jnp.int32),
                       pltpu.VMEM((C + PAD,), jnp.int32)],
    )
    counts, ranks, _ = k(classes)
    return counts, ranks


def _loss_kernel(x_ref, xb_ref, mem_ref, clsc_ref, cnt_ref, rk_ref, out_ref):
    x = x_ref[:, :]
    xb = xb_ref[:, :]
    cls_col = clsc_ref[:, :]
    counts_row = cnt_ref[:, :]
    r_col = rk_ref[:, :]

    nrm = jnp.sqrt(jnp.sum(x * x, axis=1, keepdims=True))
    feat = (x / jnp.maximum(nrm, 1e-12)).astype(_bf16)

    cls_iota = jax.lax.broadcasted_iota(jnp.int32, (B, C), 1)
    onehotb = (cls_col == cls_iota).astype(_bf16)
    onehotf = onehotb.astype(_f32)
    incl_col = (r_col < CAP).astype(_bf16)

    cm4_col = cls_col - (cls_col // 4) * 4
    bidx_g = jnp.where(r_col < CAP, cm4_col * CAP + r_col, C)
    gw = ((bidx_g == jax.lax.broadcasted_iota(jnp.int32, (B, C), 1))
          .astype(_bf16))
    qw = onehotb * incl_col

    G = jax.lax.dot_general(feat, xb, (((1,), (1,)), ((), ())),
                            preferred_element_type=_f32)
    Gb = G.astype(_bf16)
    eG = jnp.exp(Gb)
    TG = jax.lax.dot_general(eG, incl_col, (((1,), (0,)), ((), ())),
                             preferred_element_type=_f32)
    GEX = jax.lax.dot_general(eG, gw, (((1,), (0,)), ((), ())),
                              preferred_element_type=_f32)
    POSG = jax.lax.dot_general(Gb, qw, (((1,), (0,)), ((), ())),
                               preferred_element_type=_f32)

    colw = jax.lax.broadcasted_iota(jnp.int32, (W, 1), 0)
    lcls_w = colw // CAP
    t_w = colw - lcls_w * CAP
    gm4_w = lcls_w - (lcls_w // 4) * 4
    oc = (jax.lax.broadcasted_iota(jnp.int32, (W, CK), 0) // CAP ==
          jax.lax.broadcasted_iota(jnp.int32, (W, CK), 1))
    ocb = oc.astype(_bf16)
    ocf = oc.astype(_f32)
    kiota = jax.lax.broadcasted_iota(jnp.int32, (W, C), 1)

    T = TG
    EX = GEX
    posacc = jnp.sum(POSG * onehotf, axis=1, keepdims=True)

    for k in range(NCHUNK):
        mb = mem_ref[pl.ds(k * W, W), :]
        Mc = jax.lax.dot_general(feat, mb, (((1,), (1,)), ((), ())),
                                 preferred_element_type=_f32).astype(_bf16)
        eM = jnp.exp(Mc)

        countsc = counts_row[:, k * CK:(k + 1) * CK]
        ccol_w = jax.lax.dot_general(ocf, countsc,
                                     (((1,), (1,)), ((), ())),
                                     preferred_element_type=_f32)
        tpc_w = t_w + ccol_w.astype(jnp.int32)
        keep_w = (tpc_w < CAP).astype(_bf16)
        bidx_w = jnp.where(tpc_w < CAP, gm4_w * CAP + tpc_w, C)
        exw = (bidx_w == kiota).astype(_bf16)
        kwc = ocb * keep_w

        T = T + jax.lax.dot_general(eM, keep_w, (((1,), (0,)), ((), ())),
                                    preferred_element_type=_f32)
        EX = EX + jax.lax.dot_general(eM, exw, (((1,), (0,)), ((), ())),
                                      preferred_element_type=_f32)
        posc = jax.lax.dot_general(Mc, kwc, (((1,), (0,)), ((), ())),
                                   preferred_element_type=_f32)
        posacc = posacc + jnp.sum(
            posc * onehotf[:, k * CK:(k + 1) * CK], axis=1, keepdims=True)

    ex_i = jnp.sum(EX * onehotf, axis=1, keepdims=True)
    pos = posacc * (1.0 / CAP)
    Tn = T - ex_i
    denom = jnp.exp(pos) + Tn
    lossv = jnp.log(denom) - pos
    out_ref[:, :] = jnp.reshape(jnp.sum(lossv) * (1.0 / B), (1, 1))


def kernel(x, memory, classes):
    counts_i, ranks_i = _route(classes)
    counts_row = counts_i.astype(jnp.float32).reshape(1, C)
    r2d = ranks_i.reshape(B, 1)
    mem_flat = memory.reshape(C * CAP, D).astype(_bf16)
    xbf = x.astype(_bf16)
    cls2d = classes.reshape(B, 1)
    out = pl.pallas_call(
        _loss_kernel,
        out_shape=jax.ShapeDtypeStruct((1, 1), jnp.float32),
    )(x, xbf, mem_flat, cls2d, counts_row, r2d)
    return out[0, 0]


# speedup vs baseline: 1.5864x; 1.2547x over previous
import jax
import jax.numpy as jnp
from jax.experimental import pallas as pl

B = 1024
D = 128
C = 256
CAP = 64
TAUC = 1.0
CK = 64
NCHUNK = C // CK
W = CK * CAP

_f32 = jnp.float32
_bf16 = jnp.bfloat16


def _loss_kernel(x_ref, xb_ref, mem_ref, clsc_ref, out_ref):
    x = x_ref[:, :]
    xb = xb_ref[:, :]
    cls_col = clsc_ref[:, :]

    nrm = jnp.sqrt(jnp.sum(x * x, axis=1, keepdims=True))
    feat = (x / jnp.maximum(nrm, 1e-12)).astype(_bf16)

    cls_iota = jax.lax.broadcasted_iota(jnp.int32, (B, C), 1)
    onehotb = (cls_col == cls_iota).astype(_bf16)
    onehotf = onehotb.astype(_f32)
    counts_row = jnp.sum(onehotf, axis=0, keepdims=True)

    ii = jax.lax.broadcasted_iota(jnp.int32, (B, B), 0)
    jj = jax.lax.broadcasted_iota(jnp.int32, (B, B), 1)
    lt = (jj < ii).astype(_bf16)
    cex = jax.lax.dot_general(lt, onehotb, (((1,), (0,)), ((), ())),
                              preferred_element_type=_f32)
    r_col = jnp.sum(cex * onehotf, axis=1, keepdims=True).astype(jnp.int32)
    incl_col = (r_col < CAP).astype(_bf16)

    cm4_col = cls_col - (cls_col // 4) * 4
    bidx_g = jnp.where(r_col < CAP, cm4_col * CAP + r_col, C)
    gw = ((bidx_g == jax.lax.broadcasted_iota(jnp.int32, (B, C), 1))
          .astype(_bf16))
    qw = onehotb * incl_col

    G = jax.lax.dot_general(feat, xb, (((1,), (1,)), ((), ())),
                            preferred_element_type=_f32)
    Gb = G.astype(_bf16)
    eG = jnp.exp(Gb)
    TG = jax.lax.dot_general(eG, incl_col, (((1,), (0,)), ((), ())),
                             preferred_element_type=_f32)
    GEX = jax.lax.dot_general(eG, gw, (((1,), (0,)), ((), ())),
                              preferred_element_type=_f32)
    POSG = jax.lax.dot_general(Gb, qw, (((1,), (0,)), ((), ())),
                               preferred_element_type=_f32)

    colw = jax.lax.broadcasted_iota(jnp.int32, (W, 1), 0)
    lcls_w = colw // CAP
    t_w = colw - lcls_w * CAP
    gm4_w = lcls_w - (lcls_w // 4) * 4
    oc = (jax.lax.broadcasted_iota(jnp.int32, (W, CK), 0) // CAP ==
          jax.lax.broadcasted_iota(jnp.int32, (W, CK), 1))
    ocb = oc.astype(_bf16)
    ocf = oc.astype(_f32)
    kiota = jax.lax.broadcasted_iota(jnp.int32, (W, C), 1)

    T = TG
    EX = GEX
    posacc = jnp.sum(POSG * onehotf, axis=1, keepdims=True)

    for k in range(NCHUNK):
        mb = mem_ref[pl.ds(k * W, W), :]
        Mc = jax.lax.dot_general(feat, mb, (((1,), (1,)), ((), ())),
                                 preferred_element_type=_f32).astype(_bf16)
        eM = jnp.exp(Mc)

        countsc = counts_row[:, k * CK:(k + 1) * CK]
        ccol_w = jax.lax.dot_general(ocf, countsc,
                                     (((1,), (1,)), ((), ())),
                                     preferred_element_type=_f32)
        tpc_w = t_w + ccol_w.astype(jnp.int32)
        keep_w = (tpc_w < CAP).astype(_bf16)
        bidx_w = jnp.where(tpc_w < CAP, gm4_w * CAP + tpc_w, C)
        exw = (bidx_w == kiota).astype(_bf16)
        kwc = ocb * keep_w

        T = T + jax.lax.dot_general(eM, keep_w, (((1,), (0,)), ((), ())),
                                    preferred_element_type=_f32)
        EX = EX + jax.lax.dot_general(eM, exw, (((1,), (0,)), ((), ())),
                                      preferred_element_type=_f32)
        posc = jax.lax.dot_general(Mc, kwc, (((1,), (0,)), ((), ())),
                                   preferred_element_type=_f32)
        posacc = posacc + jnp.sum(
            posc * onehotf[:, k * CK:(k + 1) * CK], axis=1, keepdims=True)

    ex_i = jnp.sum(EX * onehotf, axis=1, keepdims=True)
    pos = posacc * (1.0 / CAP)
    Tn = T - ex_i
    denom = jnp.exp(pos) + Tn
    lossv = jnp.log(denom) - pos
    out_ref[:, :] = jnp.reshape(jnp.sum(lossv) * (1.0 / B), (1, 1))


def kernel(x, memory, classes):
    mem_flat = memory.reshape(C * CAP, D).astype(_bf16)
    xbf = x.astype(_bf16)
    cls2d = classes.reshape(B, 1)
    out = pl.pallas_call(
        _loss_kernel,
        out_shape=jax.ShapeDtypeStruct((1, 1), jnp.float32),
    )(x, xbf, mem_flat, cls2d)
    return out[0, 0]


# speedup vs baseline: 1.7496x; 1.1029x over previous
import jax
import jax.numpy as jnp
from jax.experimental import pallas as pl

B = 1024
D = 128
C = 256
CAP = 64
TAUC = 1.0
CK = 32
NCHUNK = C // CK
W = CK * CAP

_f32 = jnp.float32
_bf16 = jnp.bfloat16


def _loss_kernel(x_ref, xb_ref, mem_ref, clsc_ref, out_ref):
    x = x_ref[:, :]
    xb = xb_ref[:, :]
    cls_col = clsc_ref[:, :]

    nrm = jnp.sqrt(jnp.sum(x * x, axis=1, keepdims=True))
    feat = (x / jnp.maximum(nrm, 1e-12)).astype(_bf16)

    cls_iota = jax.lax.broadcasted_iota(jnp.int32, (B, C), 1)
    onehotb = (cls_col == cls_iota).astype(_bf16)
    onehotf = onehotb.astype(_f32)
    counts_row = jnp.sum(onehotf, axis=0, keepdims=True)

    ii = jax.lax.broadcasted_iota(jnp.int32, (B, B), 0)
    jj = jax.lax.broadcasted_iota(jnp.int32, (B, B), 1)
    lt = (jj < ii).astype(_bf16)
    cex = jax.lax.dot_general(lt, onehotb, (((1,), (0,)), ((), ())),
                              preferred_element_type=_f32)
    r_col = jnp.sum(cex * onehotf, axis=1, keepdims=True).astype(jnp.int32)
    incl_col = (r_col < CAP).astype(_bf16)

    cm4_col = cls_col - (cls_col // 4) * 4
    bidx_g = jnp.where(r_col < CAP, cm4_col * CAP + r_col, C)
    gw = ((bidx_g == jax.lax.broadcasted_iota(jnp.int32, (B, C), 1))
          .astype(_bf16))
    qw = onehotb * incl_col

    G = jax.lax.dot_general(feat, xb, (((1,), (1,)), ((), ())),
                            preferred_element_type=_f32)
    Gb = G.astype(_bf16)
    eG = jnp.exp(Gb)
    TG = jax.lax.dot_general(eG, incl_col, (((1,), (0,)), ((), ())),
                             preferred_element_type=_f32)
    GEX = jax.lax.dot_general(eG, gw, (((1,), (0,)), ((), ())),
                              preferred_element_type=_f32)
    POSG = jax.lax.dot_general(Gb, qw, (((1,), (0,)), ((), ())),
                               preferred_element_type=_f32)

    colw = jax.lax.broadcasted_iota(jnp.int32, (W, 1), 0)
    lcls_w = colw // CAP
    t_w = colw - lcls_w * CAP
    gm4_w = lcls_w - (lcls_w // 4) * 4
    oc = (jax.lax.broadcasted_iota(jnp.int32, (W, CK), 0) // CAP ==
          jax.lax.broadcasted_iota(jnp.int32, (W, CK), 1))
    ocb = oc.astype(_bf16)
    ocf = oc.astype(_f32)
    kiota = jax.lax.broadcasted_iota(jnp.int32, (W, C), 1)

    T = TG
    EX = GEX
    posacc = jnp.sum(POSG * onehotf, axis=1, keepdims=True)

    for k in range(NCHUNK):
        mb = mem_ref[pl.ds(k * W, W), :]
        Mc = jax.lax.dot_general(feat, mb, (((1,), (1,)), ((), ())),
                                 preferred_element_type=_f32).astype(_bf16)
        eM = jnp.exp(Mc)

        countsc = counts_row[:, k * CK:(k + 1) * CK]
        ccol_w = jax.lax.dot_general(ocf, countsc,
                                     (((1,), (1,)), ((), ())),
                                     preferred_element_type=_f32)
        tpc_w = t_w + ccol_w.astype(jnp.int32)
        keep_w = (tpc_w < CAP).astype(_bf16)
        bidx_w = jnp.where(tpc_w < CAP, gm4_w * CAP + tpc_w, C)
        exw = (bidx_w == kiota).astype(_bf16)
        kwc = ocb * keep_w

        T = T + jax.lax.dot_general(eM, keep_w, (((1,), (0,)), ((), ())),
                                    preferred_element_type=_f32)
        EX = EX + jax.lax.dot_general(eM, exw, (((1,), (0,)), ((), ())),
                                      preferred_element_type=_f32)
        posc = jax.lax.dot_general(Mc, kwc, (((1,), (0,)), ((), ())),
                                   preferred_element_type=_f32)
        posacc = posacc + jnp.sum(
            posc * onehotf[:, k * CK:(k + 1) * CK], axis=1, keepdims=True)

    ex_i = jnp.sum(EX * onehotf, axis=1, keepdims=True)
    pos = posacc * (1.0 / CAP)
    Tn = T - ex_i
    denom = jnp.exp(pos) + Tn
    lossv = jnp.log(denom) - pos
    out_ref[:, :] = jnp.reshape(jnp.sum(lossv) * (1.0 / B), (1, 1))


def kernel(x, memory, classes):
    mem_flat = memory.reshape(C * CAP, D).astype(_bf16)
    xbf = x.astype(_bf16)
    cls2d = classes.reshape(B, 1)
    out = pl.pallas_call(
        _loss_kernel,
        out_shape=jax.ShapeDtypeStruct((1, 1), jnp.float32),
    )(x, xbf, mem_flat, cls2d)
    return out[0, 0]
